# baseline (speedup 1.0000x reference)
import functools

import jax
import jax.numpy as jnp
from jax import lax
from jax.experimental import pallas as pl
from jax.experimental.pallas import tpu as pltpu
from jax.experimental.pallas import tpu_sc as plsc

D_MODEL = 1024
N_TOK = 16384
NC, NS = 2, 16
NW = NC * NS
BPW = N_TOK // NW

C1, C2 = 128, 128
N1, N2 = BPW // C1, BPW // C2
CAP = BPW + 16


@functools.cache
def _build_sc_gather():
    mesh = plsc.VectorSubcoreMesh(
        core_axis_name="c", subcore_axis_name="s",
        num_cores=NC, num_subcores=NS)

    @functools.partial(
        pl.kernel,
        out_type=(
            jax.ShapeDtypeStruct((N_TOK, D_MODEL), jnp.float32),
            jax.ShapeDtypeStruct((N_TOK, 256), jnp.float32),
            jax.ShapeDtypeStruct((N_TOK, 128), jnp.float32),
        ),
        mesh=mesh,
        compiler_params=pltpu.CompilerParams(needs_layout_passes=False),
        scratch_types=[
            pltpu.VMEM((BPW,), jnp.int32),
            pltpu.VMEM((N1, C1), jnp.int32),
            pltpu.VMEM((N2, C2), jnp.int32),
            pltpu.VMEM((CAP,), jnp.int32),
            pltpu.VMEM((CAP,), jnp.int32),
            pltpu.VMEM((2, C1, 256), jnp.float32),
            pltpu.VMEM((2, C2, 128), jnp.float32),
            pltpu.VMEM((16, D_MODEL), jnp.float32),
            pltpu.SemaphoreType.DMA,
            pltpu.SemaphoreType.DMA,
            pltpu.SemaphoreType.DMA,
            pltpu.SemaphoreType.DMA,
            pltpu.SemaphoreType.DMA,
            pltpu.SemaphoreType.DMA,
            pltpu.SemaphoreType.DMA,
            pltpu.SemaphoreType.DMA,
            pltpu.SemaphoreType.DMA,
        ],
    )
    def _sc_gather(ids_hbm, emb0, emb1, emb2, out0_hbm, g1_hbm, g2_hbm,
                   ids_v, idx1_v, idx2_v, cid_v, cpos_v, b1, b2, b0,
                   g1a, g1b, s1a, s1b, g2a, g2b, s2a, s2b, c0sem):
        wid = lax.axis_index("s") * NC + lax.axis_index("c")
        base = wid * BPW
        pltpu.sync_copy(ids_hbm.at[pl.ds(base, BPW)], ids_v)

        lane = lax.iota(jnp.int32, 16)

        def compute_idx(i, carry):
            cnt, pad_id, pad_pos = carry
            v = ids_v[pl.ds(i * 16, 16)]
            idx1_v[i // (C1 // 16), pl.ds((i % (C1 // 16)) * 16, 16)] = (
                jnp.clip(v - 20000, 0, 39999))
            idx2_v[i // (C2 // 16), pl.ds((i % (C2 // 16)) * 16, 16)] = (
                jnp.right_shift(jnp.clip(v - 60000, 0, 39999), 1))
            m0 = v < 20000
            ids0 = jnp.clip(v, 0, 19999)
            pos = base + i * 16 + lane
            plsc.store_compressed(cid_v.at[pl.ds(cnt, 16)], ids0, mask=m0)
            plsc.store_compressed(cpos_v.at[pl.ds(cnt, 16)], pos, mask=m0)
            npop = jnp.max(plsc.all_reduce_population_count(m0))
            mpos = jnp.max(jnp.where(m0, pos, -1))
            mid = jnp.max(jnp.where(jnp.where(m0, pos, -1) == mpos, ids0, -1))
            has = mpos >= 0
            return (cnt + npop,
                    jnp.where(has, mid, pad_id),
                    jnp.where(has, mpos, pad_pos))

        cnt, pad_id, pad_pos = lax.fori_loop(
            0, BPW // 16, compute_idx, (0, 0, 0))
        cid_v[pl.ds(cnt, 16)] = jnp.full((16,), pad_id, jnp.int32)
        cpos_v[pl.ds(cnt, 16)] = jnp.full((16,), pad_pos, jnp.int32)

        def fire_g1(c):
            return pltpu.async_copy(
                emb1.at[idx1_v.at[c]], b1.at[c % 2], (g1a, g1b)[c % 2])

        def fire_g2(c):
            return pltpu.async_copy(
                emb2.at[idx2_v.at[c]], b2.at[c % 2], (g2a, g2b)[c % 2])

        def fire_s1(c):
            return pltpu.async_copy(
                b1.at[c % 2], g1_hbm.at[pl.ds(base + c * C1, C1)],
                (s1a, s1b)[c % 2])

        def fire_s2(c):
            return pltpu.async_copy(
                b2.at[c % 2], g2_hbm.at[pl.ds(base + c * C2, C2)],
                (s2a, s2b)[c % 2])

        ga1 = [fire_g1(0), fire_g1(1)]
        ga2 = [fire_g2(0), fire_g2(1)]
        st1 = {}
        st2 = {}
        for c in range(N1):
            ga1[c % 2].wait()
            st1[c] = fire_s1(c)
            ga2[c % 2].wait()
            st2[c] = fire_s2(c)
            if c + 2 < N1:
                st1[c].wait()
                ga1[c % 2] = fire_g1(c + 2)
                st2[c].wait()
                ga2[c % 2] = fire_g2(c + 2)
        for c in (N1 - 2, N1 - 1):
            st1[c].wait()
            st2[c].wait()

        nch = jnp.right_shift(cnt + 15, 4)

        def c0_chunk(c, carry):
            iv = cid_v[pl.ds(c * 16, 16)]
            pv = cpos_v[pl.ds(c * 16, 16)]
            pltpu.async_copy(emb0.at[iv], b0, c0sem).wait()
            pltpu.async_copy(b0, out0_hbm.at[pv], c0sem).wait()
            return carry

        lax.fori_loop(0, nch, c0_chunk, 0)

    return _sc_gather


BT = 512


def _tc_body(ids_ref, p0_ref, g1_ref, g2_ref, w1_ref, w2_ref, out_ref):
    ids = ids_ref[...]
    m1 = (ids >= 20000) & (ids < 60000)
    m2 = ids >= 60000
    g1 = jnp.where(m1, g1_ref[...], 0.0)
    lane = lax.broadcasted_iota(jnp.int32, (BT, 128), 1)
    parity = (ids - 60000) & 1
    half_ok = (lane >= 64) == (parity == 1)
    g2 = jnp.where(m2 & half_ok, g2_ref[...], 0.0)
    w2 = w2_ref[...]
    w2x = jnp.concatenate([w2, w2], axis=0)
    acc = jnp.dot(g1, w1_ref[...], preferred_element_type=jnp.float32)
    acc = acc + jnp.dot(g2, w2x, preferred_element_type=jnp.float32)
    out_ref[...] = jnp.where(ids < 20000, p0_ref[...], acc)


_tc_combine = pl.pallas_call(
    _tc_body,
    grid=(N_TOK // BT,),
    in_specs=[
        pl.BlockSpec((BT, 1), lambda i: (i, 0)),
        pl.BlockSpec((BT, D_MODEL), lambda i: (i, 0)),
        pl.BlockSpec((BT, 256), lambda i: (i, 0)),
        pl.BlockSpec((BT, 128), lambda i: (i, 0)),
        pl.BlockSpec((256, D_MODEL), lambda i: (0, 0)),
        pl.BlockSpec((64, D_MODEL), lambda i: (0, 0)),
    ],
    out_specs=pl.BlockSpec((BT, D_MODEL), lambda i: (i, 0)),
    out_shape=jax.ShapeDtypeStruct((N_TOK, D_MODEL), jnp.float32),
    input_output_aliases={1: 0},
)


def kernel(input_ids, emb0, emb1, emb2, W1, W2):
    ids = input_ids.reshape(-1).astype(jnp.int32)
    emb2r = emb2.reshape(20000, 128)
    out0, g1, g2 = _build_sc_gather()(ids, emb0, emb1, emb2r)
    out = _tc_combine(ids.reshape(N_TOK, 1), out0, g1, g2, W1, W2)
    return out.reshape(input_ids.shape + (D_MODEL,))
